# baseline (speedup 1.0000x reference)
import jax
import jax.numpy as jnp
from jax import lax
from jax.experimental import pallas as pl
from jax.experimental.pallas import tpu as pltpu
from jax.experimental.pallas import tpu_sc as plsc

N = 10000
E = 320000
NPAD = 10240
LANES = 16
NC, NS = 2, 16
NW = NC * NS
EPW = E // NW
BLK = 1024
D0 = 192
NCLS = 16


def _sc_mark_body(dst_hbm, out_hbm, idx_v, mark_v):
    wid = lax.axis_index("s") * NC + lax.axis_index("c")
    pltpu.sync_copy(dst_hbm.at[pl.ds(wid * EPW, EPW)], idx_v)
    zeros = jnp.zeros((LANES,), jnp.float32)

    def zero_body(j, carry):
        mark_v[pl.ds(j * LANES, LANES)] = zeros
        return carry

    lax.fori_loop(0, NPAD // LANES, zero_body, 0)
    ones = jnp.ones((LANES,), jnp.float32)

    def scat_body(j, carry):
        idx16 = idx_v[pl.ds(j * LANES, LANES)]
        plsc.store_scatter(mark_v, [idx16], ones)
        return carry

    lax.fori_loop(0, EPW // LANES, scat_body, 0)
    pltpu.sync_copy(mark_v, out_hbm.at[pl.ds(wid * NPAD, NPAD)])


def _sc_marks(dst):
    mesh = plsc.VectorSubcoreMesh(core_axis_name="c", subcore_axis_name="s")
    return pl.kernel(
        _sc_mark_body,
        out_type=jax.ShapeDtypeStruct((NW * NPAD,), jnp.float32),
        mesh=mesh,
        compiler_params=pltpu.CompilerParams(needs_layout_passes=False),
        scratch_types=[
            pltpu.VMEM((EPW,), jnp.int32),
            pltpu.VMEM((NPAD,), jnp.float32),
        ],
    )(dst)


def _dot_nt(a, b):
    return lax.dot_general(a, b, (((1,), (1,)), ((), ())),
                           preferred_element_type=jnp.float32)


def _tc_dense_body(x_ref, mk_ref, w0_ref, wih0_ref, b0_ref,
                   w1_ref, wih1_ref, b1_ref, w2_ref, wih2_ref, b2_ref,
                   out_ref):
    mask = (jnp.max(mk_ref[...], axis=1, keepdims=True) > 0.0).astype(jnp.float32)

    def lstm(u, wih_ref, b_ref, H):
        gi = jax.nn.sigmoid(_dot_nt(u, wih_ref[0:H, :]) + b_ref[0:1, :])
        gg = jnp.tanh(_dot_nt(u, wih_ref[2 * H:3 * H, :]) + b_ref[2:3, :])
        go = jax.nn.sigmoid(_dot_nt(u, wih_ref[3 * H:4 * H, :]) + b_ref[3:4, :])
        return go * jnp.tanh(gi * gg)

    u = jnp.maximum(_dot_nt(x_ref[...], w0_ref[...]), 0.0) * mask
    h = lstm(u, wih0_ref, b0_ref, D0)
    u = jnp.maximum(_dot_nt(h, w1_ref[...]), 0.0) * mask
    h = lstm(u, wih1_ref, b1_ref, D0)
    u = _dot_nt(h, w2_ref[...]) * mask
    out_ref[...] = lstm(u, wih2_ref, b2_ref, NCLS)


def _tc_specs():
    full = lambda shape: pl.BlockSpec(shape, lambda i: (0, 0))
    return [pl.BlockSpec((BLK, 128), lambda i: (i, 0)),
            pl.BlockSpec((BLK, NW), lambda i: (i, 0)),
            full((D0, 128)), full((4 * D0, D0)), full((4, D0)),
            full((D0, D0)), full((4 * D0, D0)), full((4, D0)),
            full((NCLS, D0)), full((4 * NCLS, NCLS)), full((4, NCLS))]


def kernel(x, edge_index, W0, A0, Wih0, bih0, bhh0, W1, A1, Wih1, bih1, bhh1,
           W2, A2, Wih2, bih2, bhh2):
    dst = edge_index[1].astype(jnp.int32)
    marks = _sc_marks(dst)
    mk = marks.reshape(NW, NPAD).T
    xp = jnp.pad(x, ((0, NPAD - N), (0, 0)))

    out = pl.pallas_call(
        _tc_dense_body,
        grid=(NPAD // BLK,),
        in_specs=_tc_specs(),
        out_specs=pl.BlockSpec((BLK, NCLS), lambda i: (i, 0)),
        out_shape=jax.ShapeDtypeStruct((NPAD, NCLS), jnp.float32),
    )(xp, mk,
      W0.reshape(D0, 128), Wih0, (bih0 + bhh0).reshape(4, D0),
      W1.reshape(D0, D0), Wih1, (bih1 + bhh1).reshape(4, D0),
      W2.reshape(NCLS, D0), Wih2, (bih2 + bhh2).reshape(4, NCLS))
    return out[:N]


# speedup vs baseline: 1.1793x; 1.1793x over previous
import jax
import jax.numpy as jnp
from jax import lax
from jax.experimental import pallas as pl
from jax.experimental.pallas import tpu as pltpu
from jax.experimental.pallas import tpu_sc as plsc

N = 10000
E = 320000
NPAD = 10240
LANES = 16
NC, NS = 2, 16
NW = NC * NS
EPW = E // NW
NBLK = E // 128
BPW = -(-NBLK // NW)
BLK = 2048
D0 = 192
NCLS = 16


def _sc_mark_body(edges_hbm, out_hbm, idx_v, mark_v):
    wid = lax.axis_index("s") * NC + lax.axis_index("c")
    start_blk = jnp.minimum(wid * BPW, NBLK - BPW)
    pltpu.sync_copy(edges_hbm.at[pl.ds(start_blk * 256, BPW * 256)], idx_v)
    zeros = jnp.zeros((LANES,), jnp.float32)

    def zero_body(j, carry):
        mark_v[pl.ds(j * LANES, LANES)] = zeros
        return carry

    lax.fori_loop(0, NPAD // LANES, zero_body, 0)
    ones = jnp.ones((LANES,), jnp.float32)

    def scat_body(b, carry):
        for k in range(128 // LANES):
            idx16 = idx_v[pl.ds(b * 256 + 128 + k * LANES, LANES)]
            plsc.store_scatter(mark_v, [idx16], ones)
        return carry

    lax.fori_loop(0, BPW, scat_body, 0)
    pltpu.sync_copy(mark_v, out_hbm.at[pl.ds(wid * NPAD, NPAD)])


def _sc_marks(edges_flat):
    mesh = plsc.VectorSubcoreMesh(core_axis_name="c", subcore_axis_name="s")
    return pl.kernel(
        _sc_mark_body,
        out_type=jax.ShapeDtypeStruct((NW * NPAD,), jnp.float32),
        mesh=mesh,
        compiler_params=pltpu.CompilerParams(needs_layout_passes=False),
        scratch_types=[
            pltpu.VMEM((BPW * 256,), jnp.int32),
            pltpu.VMEM((NPAD,), jnp.float32),
        ],
    )(edges_flat)


def _dot_nt(a, b):
    return lax.dot_general(a, b, (((1,), (1,)), ((), ())),
                           preferred_element_type=jnp.float32)


def _tc_dense_body(x_ref, mk_ref, w0_ref, wih0_ref, b0_ref,
                   w1_ref, wih1_ref, b1_ref, w2_ref, wih2_ref, b2_ref,
                   out_ref):
    mkt = jnp.transpose(mk_ref[...], (1, 0))
    mask = (jnp.max(mkt, axis=1, keepdims=True) > 0.0).astype(jnp.float32)

    def sigmoid(x):
        return 0.5 * jnp.tanh(0.5 * x) + 0.5

    def lstm(u, wih_ref, b_ref, H):
        gi = sigmoid(_dot_nt(u, wih_ref[0:H, :]) + b_ref[0:1, :])
        gg = jnp.tanh(_dot_nt(u, wih_ref[2 * H:3 * H, :]) + b_ref[2:3, :])
        go = sigmoid(_dot_nt(u, wih_ref[3 * H:4 * H, :]) + b_ref[3:4, :])
        return go * jnp.tanh(gi * gg)

    u = jnp.maximum(_dot_nt(x_ref[...], w0_ref[...]), 0.0) * mask
    h = lstm(u, wih0_ref, b0_ref, D0)
    u = jnp.maximum(_dot_nt(h, w1_ref[...]), 0.0) * mask
    h = lstm(u, wih1_ref, b1_ref, D0)
    u = _dot_nt(h, w2_ref[...]) * mask
    out_ref[...] = lstm(u, wih2_ref, b2_ref, NCLS)


def _tc_specs():
    full = lambda shape: pl.BlockSpec(shape, lambda i: (0, 0))
    return [pl.BlockSpec((BLK, 128), lambda i: (i, 0)),
            pl.BlockSpec((NW, BLK), lambda i: (0, i)),
            full((D0, 128)), full((4 * D0, D0)), full((4, D0)),
            full((D0, D0)), full((4 * D0, D0)), full((4, D0)),
            full((NCLS, D0)), full((4 * NCLS, NCLS)), full((4, NCLS))]


def kernel(x, edge_index, W0, A0, Wih0, bih0, bhh0, W1, A1, Wih1, bih1, bhh1,
           W2, A2, Wih2, bih2, bhh2):
    ei = edge_index.reshape(2, NBLK, 128).transpose(1, 0, 2).reshape(2 * E)
    marks = _sc_marks(ei)
    mk = marks.reshape(NW, NPAD)

    out = pl.pallas_call(
        _tc_dense_body,
        grid=(pl.cdiv(N, BLK),),
        in_specs=_tc_specs(),
        out_specs=pl.BlockSpec((BLK, NCLS), lambda i: (i, 0)),
        out_shape=jax.ShapeDtypeStruct((N, NCLS), jnp.float32),
    )(x, mk,
      W0.reshape(D0, 128), Wih0, (bih0 + bhh0).reshape(4, D0),
      W1.reshape(D0, D0), Wih1, (bih1 + bhh1).reshape(4, D0),
      W2.reshape(NCLS, D0), Wih2, (bih2 + bhh2).reshape(4, NCLS))
    return out


# speedup vs baseline: 1.3227x; 1.1216x over previous
import jax
import jax.numpy as jnp
from jax import lax
from jax.experimental import pallas as pl
from jax.experimental.pallas import tpu as pltpu
from jax.experimental.pallas import tpu_sc as plsc

N = 10000
E = 320000
NPAD = 10240
LANES = 16
NC, NS = 2, 16
NW = NC * NS
EPW = E // NW
NBLK = E // 128
BPW = -(-NBLK // NW)
BLK = 2048
D0 = 192
NCLS = 16


def _sc_mark_body(edges_hbm, out_hbm, idx_v, mark_v):
    wid = lax.axis_index("s") * NC + lax.axis_index("c")
    pltpu.sync_copy(edges_hbm.at[pl.ds(E + wid * EPW, EPW)], idx_v)
    zeros = jnp.zeros((LANES,), jnp.float32)

    def zero_body(j, carry):
        mark_v[pl.ds(j * LANES, LANES)] = zeros
        return carry

    lax.fori_loop(0, NPAD // LANES, zero_body, 0)
    ones = jnp.ones((LANES,), jnp.float32)

    def scat_body(j, carry):
        idx16 = idx_v[pl.ds(j * LANES, LANES)]
        plsc.store_scatter(mark_v, [idx16], ones)
        return carry

    lax.fori_loop(0, EPW // LANES, scat_body, 0)
    pltpu.sync_copy(mark_v, out_hbm.at[pl.ds(wid * NPAD, NPAD)])


def _sc_marks(edges_flat):
    mesh = plsc.VectorSubcoreMesh(core_axis_name="c", subcore_axis_name="s")
    return pl.kernel(
        _sc_mark_body,
        out_type=jax.ShapeDtypeStruct((NW * NPAD,), jnp.float32),
        mesh=mesh,
        compiler_params=pltpu.CompilerParams(needs_layout_passes=False),
        scratch_types=[
            pltpu.VMEM((EPW,), jnp.int32),
            pltpu.VMEM((NPAD,), jnp.float32),
        ],
    )(edges_flat)


def _dot_nt(a, b):
    return lax.dot_general(a, b, (((1,), (1,)), ((), ())),
                           preferred_element_type=jnp.float32)


def _tc_dense_body(x_ref, mk_ref, w0_ref, wih0_ref, b0_ref,
                   w1_ref, wih1_ref, b1_ref, w2_ref, wih2_ref, b2_ref,
                   out_ref):
    mkt = jnp.transpose(mk_ref[...], (1, 0))
    mask = (jnp.max(mkt, axis=1, keepdims=True) > 0.0).astype(jnp.float32)

    def sigmoid(x):
        return 0.5 * jnp.tanh(0.5 * x) + 0.5

    def lstm(u, wih_ref, b_ref, H):
        gi = sigmoid(_dot_nt(u, wih_ref[0:H, :]) + b_ref[0:1, :])
        gg = jnp.tanh(_dot_nt(u, wih_ref[2 * H:3 * H, :]) + b_ref[2:3, :])
        go = sigmoid(_dot_nt(u, wih_ref[3 * H:4 * H, :]) + b_ref[3:4, :])
        return go * jnp.tanh(gi * gg)

    u = jnp.maximum(_dot_nt(x_ref[...], w0_ref[...]), 0.0) * mask
    h = lstm(u, wih0_ref, b0_ref, D0)
    u = jnp.maximum(_dot_nt(h, w1_ref[...]), 0.0) * mask
    h = lstm(u, wih1_ref, b1_ref, D0)
    u = _dot_nt(h, w2_ref[...]) * mask
    out_ref[...] = lstm(u, wih2_ref, b2_ref, NCLS)


def _tc_specs():
    full = lambda shape: pl.BlockSpec(shape, lambda i: (0, 0))
    return [pl.BlockSpec((BLK, 128), lambda i: (i, 0)),
            pl.BlockSpec((NW, BLK), lambda i: (0, i)),
            full((D0, 128)), full((4 * D0, D0)), full((4, D0)),
            full((D0, D0)), full((4 * D0, D0)), full((4, D0)),
            full((NCLS, D0)), full((4 * NCLS, NCLS)), full((4, NCLS))]


def kernel(x, edge_index, W0, A0, Wih0, bih0, bhh0, W1, A1, Wih1, bih1, bhh1,
           W2, A2, Wih2, bih2, bhh2):
    marks = _sc_marks(edge_index.reshape(2 * E))
    mk = marks.reshape(NW, NPAD)

    out = pl.pallas_call(
        _tc_dense_body,
        grid=(pl.cdiv(N, BLK),),
        in_specs=_tc_specs(),
        out_specs=pl.BlockSpec((BLK, NCLS), lambda i: (i, 0)),
        out_shape=jax.ShapeDtypeStruct((N, NCLS), jnp.float32),
    )(x, mk,
      W0.reshape(D0, 128), Wih0, (bih0 + bhh0).reshape(4, D0),
      W1.reshape(D0, D0), Wih1, (bih1 + bhh1).reshape(4, D0),
      W2.reshape(NCLS, D0), Wih2, (bih2 + bhh2).reshape(4, NCLS))
    return out


# speedup vs baseline: 1.5506x; 1.1722x over previous
import jax
import jax.numpy as jnp
from jax import lax
from jax.experimental import pallas as pl
from jax.experimental.pallas import tpu as pltpu
from jax.experimental.pallas import tpu_sc as plsc

N = 10000
E = 320000
NPAD = 10240
LANES = 16
NC, NS = 2, 16
NW = NC * NS
EPW = E // NW
NBLK = E // 128
BPW = -(-NBLK // NW)
BLK = 2048
D0 = 192
NCLS = 16


def _sc_mark_body(edges_hbm, out_hbm, idx_v, mark_v):
    wid = lax.axis_index("s") * NC + lax.axis_index("c")
    pltpu.sync_copy(edges_hbm.at[pl.ds(E + wid * EPW, EPW)], idx_v)
    zeros = jnp.zeros((LANES,), jnp.float32)

    def zero_body(j, carry):
        mark_v[pl.ds(j * LANES, LANES)] = zeros
        return carry

    lax.fori_loop(0, NPAD // LANES, zero_body, 0)
    ones = jnp.ones((LANES,), jnp.float32)

    def scat_body(j, carry):
        idx16 = idx_v[pl.ds(j * LANES, LANES)]
        plsc.store_scatter(mark_v, [idx16], ones)
        return carry

    lax.fori_loop(0, EPW // LANES, scat_body, 0)
    pltpu.sync_copy(mark_v, out_hbm.at[pl.ds(wid * NPAD, NPAD)])


def _sc_marks(edges_flat):
    mesh = plsc.VectorSubcoreMesh(core_axis_name="c", subcore_axis_name="s")
    return pl.kernel(
        _sc_mark_body,
        out_type=jax.ShapeDtypeStruct((NW * NPAD,), jnp.float32),
        mesh=mesh,
        compiler_params=pltpu.CompilerParams(needs_layout_passes=False),
        scratch_types=[
            pltpu.VMEM((EPW,), jnp.int32),
            pltpu.VMEM((NPAD,), jnp.float32),
        ],
    )(edges_flat)


def _dot_nt(a, b):
    return lax.dot_general(a, b, (((1,), (1,)), ((), ())),
                           preferred_element_type=jnp.float32)


def _dot_nn(a, b):
    return lax.dot_general(a, b, (((1,), (0,)), ((), ())),
                           preferred_element_type=jnp.float32)


def _tc_dense_body(x_ref, mk_ref, w0_ref, wih0_ref, b0_ref,
                   w1_ref, wih1_ref, b1_ref, w2_ref, wih2_ref, b2_ref,
                   out_ref):
    mask = (jnp.max(mk_ref[...], axis=0, keepdims=True) > 0.0).astype(jnp.float32)

    def sigmoid(x):
        return 0.5 * jnp.tanh(0.5 * x) + 0.5

    def lstm(u, wih_ref, b_ref, H):
        gi = sigmoid(_dot_nn(wih_ref[0:H, :], u) + b_ref[:, 0:1])
        gg = jnp.tanh(_dot_nn(wih_ref[2 * H:3 * H, :], u) + b_ref[:, 2:3])
        go = sigmoid(_dot_nn(wih_ref[3 * H:4 * H, :], u) + b_ref[:, 3:4])
        return go * jnp.tanh(gi * gg)

    u = jnp.maximum(_dot_nt(w0_ref[...], x_ref[...]), 0.0) * mask
    h = lstm(u, wih0_ref, b0_ref, D0)
    u = jnp.maximum(_dot_nn(w1_ref[...], h), 0.0) * mask
    h = lstm(u, wih1_ref, b1_ref, D0)
    u = _dot_nn(w2_ref[...], h) * mask
    out_ref[...] = lstm(u, wih2_ref, b2_ref, NCLS)


def _tc_specs():
    full = lambda shape: pl.BlockSpec(shape, lambda i: (0, 0))
    return [pl.BlockSpec((BLK, 128), lambda i: (i, 0)),
            pl.BlockSpec((NW, BLK), lambda i: (0, i)),
            full((D0, 128)), full((4 * D0, D0)), full((D0, 4)),
            full((D0, D0)), full((4 * D0, D0)), full((D0, 4)),
            full((NCLS, D0)), full((4 * NCLS, NCLS)), full((NCLS, 4))]


def kernel(x, edge_index, W0, A0, Wih0, bih0, bhh0, W1, A1, Wih1, bih1, bhh1,
           W2, A2, Wih2, bih2, bhh2):
    marks = _sc_marks(edge_index.reshape(2 * E))
    mk = marks.reshape(NW, NPAD)

    out = pl.pallas_call(
        _tc_dense_body,
        grid=(pl.cdiv(N, BLK),),
        in_specs=_tc_specs(),
        out_specs=pl.BlockSpec((NCLS, BLK), lambda i: (0, i)),
        out_shape=jax.ShapeDtypeStruct((NCLS, N), jnp.float32),
    )(x, mk,
      W0.reshape(D0, 128), Wih0, (bih0 + bhh0).reshape(4, D0).T,
      W1.reshape(D0, D0), Wih1, (bih1 + bhh1).reshape(4, D0).T,
      W2.reshape(NCLS, D0), Wih2, (bih2 + bhh2).reshape(4, NCLS).T)
    return out.T


# speedup vs baseline: 1.6419x; 1.0589x over previous
import jax
import jax.numpy as jnp
from jax import lax
from jax.experimental import pallas as pl
from jax.experimental.pallas import tpu as pltpu
from jax.experimental.pallas import tpu_sc as plsc

N = 10000
E = 320000
NPAD = 10240
LANES = 16
NC, NS = 2, 16
NW = NC * NS
EPW = E // NW
NBLK = E // 128
BPW = -(-NBLK // NW)
BLK = 2048
D0 = 192
NCLS = 16


def _sc_mark_body(edges_hbm, out_hbm, idx_v, mark_v):
    wid = lax.axis_index("s") * NC + lax.axis_index("c")
    pltpu.sync_copy(edges_hbm.at[pl.ds(E + wid * EPW, EPW)], idx_v)
    zeros = jnp.zeros((LANES,), jnp.float32)

    def zero_body(j, carry):
        mark_v[pl.ds(j * LANES, LANES)] = zeros
        return carry

    lax.fori_loop(0, NPAD // LANES, zero_body, 0)
    ones = jnp.ones((LANES,), jnp.float32)

    def scat_body(j, carry):
        idx16 = idx_v[pl.ds(j * LANES, LANES)]
        plsc.store_scatter(mark_v, [idx16], ones)
        return carry

    lax.fori_loop(0, EPW // LANES, scat_body, 0)
    pltpu.sync_copy(mark_v, out_hbm.at[pl.ds(wid * NPAD, NPAD)])


def _sc_marks(edges_flat):
    mesh = plsc.VectorSubcoreMesh(core_axis_name="c", subcore_axis_name="s")
    return pl.kernel(
        _sc_mark_body,
        out_type=jax.ShapeDtypeStruct((NW * NPAD,), jnp.float32),
        mesh=mesh,
        compiler_params=pltpu.CompilerParams(needs_layout_passes=False),
        scratch_types=[
            pltpu.VMEM((EPW,), jnp.int32),
            pltpu.VMEM((NPAD,), jnp.float32),
        ],
    )(edges_flat)


def _dot_nt(a, b):
    return lax.dot_general(a, b, (((1,), (1,)), ((), ())),
                           preferred_element_type=jnp.float32)


def _dot_nn(a, b):
    return lax.dot_general(a, b, (((1,), (0,)), ((), ())),
                           preferred_element_type=jnp.float32)


def _tc_dense_body(x_ref, mk_ref, w0_ref, wih0_ref, b0_ref,
                   w1_ref, wih1_ref, b1_ref, w2_ref, wih2_ref, b2_ref,
                   out_ref):
    mask = (jnp.max(mk_ref[...], axis=0).reshape(1, BLK) > 0.0).astype(jnp.float32)

    def sigmoid(x):
        return 0.5 * jnp.tanh(0.5 * x) + 0.5

    def lstm(u, wih_ref, b_ref, H):
        gi = sigmoid(_dot_nn(wih_ref[0:H, :], u) + b_ref[:, 0:1])
        gg = jnp.tanh(_dot_nn(wih_ref[2 * H:3 * H, :], u) + b_ref[:, 2:3])
        go = sigmoid(_dot_nn(wih_ref[3 * H:4 * H, :], u) + b_ref[:, 3:4])
        return go * jnp.tanh(gi * gg)

    u = jnp.maximum(_dot_nt(w0_ref[...], x_ref[...]), 0.0) * mask
    h = lstm(u, wih0_ref, b0_ref, D0)
    u = jnp.maximum(_dot_nn(w1_ref[...], h), 0.0) * mask
    h = lstm(u, wih1_ref, b1_ref, D0)
    u = _dot_nn(w2_ref[...], h) * mask
    out_ref[...] = lstm(u, wih2_ref, b2_ref, NCLS)


def _tc_specs():
    full = lambda shape: pl.BlockSpec(shape, lambda i: (0, 0))
    return [pl.BlockSpec((BLK, 128), lambda i: (i, 0)),
            pl.BlockSpec((NW, BLK // 128, 128), lambda i: (0, i, 0)),
            full((D0, 128)), full((4 * D0, D0)), full((D0, 4)),
            full((D0, D0)), full((4 * D0, D0)), full((D0, 4)),
            full((NCLS, D0)), full((4 * NCLS, NCLS)), full((NCLS, 4))]


def kernel(x, edge_index, W0, A0, Wih0, bih0, bhh0, W1, A1, Wih1, bih1, bhh1,
           W2, A2, Wih2, bih2, bhh2):
    marks = _sc_marks(edge_index.reshape(2 * E))
    mk = marks.reshape(NW, NPAD // 128, 128)

    out = pl.pallas_call(
        _tc_dense_body,
        grid=(pl.cdiv(N, BLK),),
        in_specs=_tc_specs(),
        out_specs=pl.BlockSpec((NCLS, BLK), lambda i: (0, i)),
        out_shape=jax.ShapeDtypeStruct((NCLS, N), jnp.float32),
    )(x, mk,
      W0.reshape(D0, 128), Wih0, (bih0 + bhh0).reshape(4, D0).T,
      W1.reshape(D0, D0), Wih1, (bih1 + bhh1).reshape(4, D0).T,
      W2.reshape(NCLS, D0), Wih2, (bih2 + bhh2).reshape(4, NCLS).T)
    return out.T


# speedup vs baseline: 1.7341x; 1.0562x over previous
import jax
import jax.numpy as jnp
from jax import lax
from jax.experimental import pallas as pl
from jax.experimental.pallas import tpu as pltpu
from jax.experimental.pallas import tpu_sc as plsc

N = 10000
E = 320000
NPAD = 10240
LANES = 16
NC, NS = 2, 16
NW = NC * NS
EPW = E // NW
NBLK = E // 128
BPW = -(-NBLK // NW)
BLK = 2048
D0 = 192
NCLS = 16


def _sc_mark_body(edges_hbm, out_hbm, idx_v, mark_v):
    wid = lax.axis_index("s") * NC + lax.axis_index("c")
    pltpu.sync_copy(edges_hbm.at[pl.ds(E + wid * EPW, EPW)], idx_v)
    zeros = jnp.zeros((LANES,), jnp.float32)

    UNROLL = 5

    def zero_body(j, carry):
        for k in range(UNROLL):
            mark_v[pl.ds((j * UNROLL + k) * LANES, LANES)] = zeros
        return carry

    lax.fori_loop(0, NPAD // LANES // UNROLL, zero_body, 0)
    ones = jnp.ones((LANES,), jnp.float32)

    def scat_body(j, carry):
        for k in range(UNROLL):
            idx16 = idx_v[pl.ds((j * UNROLL + k) * LANES, LANES)]
            plsc.store_scatter(mark_v, [idx16], ones)
        return carry

    lax.fori_loop(0, EPW // LANES // UNROLL, scat_body, 0)
    pltpu.sync_copy(mark_v, out_hbm.at[pl.ds(wid * NPAD, NPAD)])


def _sc_marks(edges_flat):
    mesh = plsc.VectorSubcoreMesh(core_axis_name="c", subcore_axis_name="s")
    return pl.kernel(
        _sc_mark_body,
        out_type=jax.ShapeDtypeStruct((NW * NPAD,), jnp.float32),
        mesh=mesh,
        compiler_params=pltpu.CompilerParams(needs_layout_passes=False),
        scratch_types=[
            pltpu.VMEM((EPW,), jnp.int32),
            pltpu.VMEM((NPAD,), jnp.float32),
        ],
    )(edges_flat)


def _dot_nt(a, b):
    return lax.dot_general(a, b, (((1,), (1,)), ((), ())),
                           preferred_element_type=jnp.float32)


def _dot_nn(a, b):
    return lax.dot_general(a, b, (((1,), (0,)), ((), ())),
                           preferred_element_type=jnp.float32)


def _tc_dense_body(x_ref, mk_ref, w0_ref, wih0_ref, b0_ref,
                   w1_ref, wih1_ref, b1_ref, w2_ref, wih2_ref, b2_ref,
                   out_ref):
    mask = (jnp.max(mk_ref[...], axis=0).reshape(1, BLK) > 0.0).astype(jnp.float32)

    def sigmoid(x):
        return 0.5 * jnp.tanh(0.5 * x) + 0.5

    def lstm(u, wih_ref, b_ref, H):
        gi = sigmoid(_dot_nn(wih_ref[0:H, :], u) + b_ref[:, 0:1])
        gg = jnp.tanh(_dot_nn(wih_ref[2 * H:3 * H, :], u) + b_ref[:, 2:3])
        go = sigmoid(_dot_nn(wih_ref[3 * H:4 * H, :], u) + b_ref[:, 3:4])
        return go * jnp.tanh(gi * gg)

    u = jnp.maximum(_dot_nt(w0_ref[...], x_ref[...]), 0.0) * mask
    h = lstm(u, wih0_ref, b0_ref, D0)
    u = jnp.maximum(_dot_nn(w1_ref[...], h), 0.0) * mask
    h = lstm(u, wih1_ref, b1_ref, D0)
    u = _dot_nn(w2_ref[...], h) * mask
    out_ref[...] = lstm(u, wih2_ref, b2_ref, NCLS)


def _tc_specs():
    full = lambda shape: pl.BlockSpec(shape, lambda i: (0, 0))
    return [pl.BlockSpec((BLK, 128), lambda i: (i, 0)),
            pl.BlockSpec((NW, BLK // 128, 128), lambda i: (0, i, 0)),
            full((D0, 128)), full((4 * D0, D0)), full((D0, 4)),
            full((D0, D0)), full((4 * D0, D0)), full((D0, 4)),
            full((NCLS, D0)), full((4 * NCLS, NCLS)), full((NCLS, 4))]


def kernel(x, edge_index, W0, A0, Wih0, bih0, bhh0, W1, A1, Wih1, bih1, bhh1,
           W2, A2, Wih2, bih2, bhh2):
    marks = _sc_marks(edge_index.reshape(2 * E))
    mk = marks.reshape(NW, NPAD // 128, 128)

    out = pl.pallas_call(
        _tc_dense_body,
        grid=(pl.cdiv(N, BLK),),
        in_specs=_tc_specs(),
        out_specs=pl.BlockSpec((NCLS, BLK), lambda i: (0, i)),
        out_shape=jax.ShapeDtypeStruct((NCLS, N), jnp.float32),
    )(x, mk,
      W0.reshape(D0, 128), Wih0, (bih0 + bhh0).reshape(4, D0).T,
      W1.reshape(D0, D0), Wih1, (bih1 + bhh1).reshape(4, D0).T,
      W2.reshape(NCLS, D0), Wih2, (bih2 + bhh2).reshape(4, NCLS).T)
    return out.T


# speedup vs baseline: 1.7396x; 1.0031x over previous
import jax
import jax.numpy as jnp
from jax import lax
from jax.experimental import pallas as pl
from jax.experimental.pallas import tpu as pltpu
from jax.experimental.pallas import tpu_sc as plsc

N = 10000
E = 320000
NPAD = 10240
LANES = 16
NC, NS = 2, 16
NW = NC * NS
EPW = E // NW
NBLK = E // 128
BPW = -(-NBLK // NW)
BLK = 5120
D0 = 192
NCLS = 16


def _sc_mark_body(edges_hbm, out_hbm, idx_v, mark_v):
    wid = lax.axis_index("s") * NC + lax.axis_index("c")
    pltpu.sync_copy(edges_hbm.at[pl.ds(E + wid * EPW, EPW)], idx_v)
    zeros = jnp.zeros((LANES,), jnp.float32)

    UNROLL = 5

    def zero_body(j, carry):
        for k in range(UNROLL):
            mark_v[pl.ds((j * UNROLL + k) * LANES, LANES)] = zeros
        return carry

    lax.fori_loop(0, NPAD // LANES // UNROLL, zero_body, 0)
    ones = jnp.ones((LANES,), jnp.float32)

    def scat_body(j, carry):
        for k in range(UNROLL):
            idx16 = idx_v[pl.ds((j * UNROLL + k) * LANES, LANES)]
            plsc.store_scatter(mark_v, [idx16], ones)
        return carry

    lax.fori_loop(0, EPW // LANES // UNROLL, scat_body, 0)
    pltpu.sync_copy(mark_v, out_hbm.at[pl.ds(wid * NPAD, NPAD)])


def _sc_marks(edges_flat):
    mesh = plsc.VectorSubcoreMesh(core_axis_name="c", subcore_axis_name="s")
    return pl.kernel(
        _sc_mark_body,
        out_type=jax.ShapeDtypeStruct((NW * NPAD,), jnp.float32),
        mesh=mesh,
        compiler_params=pltpu.CompilerParams(needs_layout_passes=False),
        scratch_types=[
            pltpu.VMEM((EPW,), jnp.int32),
            pltpu.VMEM((NPAD,), jnp.float32),
        ],
    )(edges_flat)


def _dot_nt(a, b):
    return lax.dot_general(a, b, (((1,), (1,)), ((), ())),
                           preferred_element_type=jnp.float32)


def _dot_nn(a, b):
    return lax.dot_general(a, b, (((1,), (0,)), ((), ())),
                           preferred_element_type=jnp.float32)


def _tc_dense_body(x_ref, mk_ref, w0_ref, wih0_ref, b0_ref,
                   w1_ref, wih1_ref, b1_ref, w2_ref, wih2_ref, b2_ref,
                   out_ref):
    mask = (jnp.max(mk_ref[...], axis=0).reshape(1, BLK) > 0.0).astype(jnp.float32)

    def sigmoid(x):
        return 0.5 * jnp.tanh(0.5 * x) + 0.5

    def lstm(u, wih_ref, b_ref, H):
        gi = sigmoid(_dot_nn(wih_ref[0:H, :], u) + b_ref[:, 0:1])
        gg = jnp.tanh(_dot_nn(wih_ref[2 * H:3 * H, :], u) + b_ref[:, 2:3])
        go = sigmoid(_dot_nn(wih_ref[3 * H:4 * H, :], u) + b_ref[:, 3:4])
        return go * jnp.tanh(gi * gg)

    u = jnp.maximum(_dot_nt(w0_ref[...], x_ref[...]), 0.0) * mask
    h = lstm(u, wih0_ref, b0_ref, D0)
    u = jnp.maximum(_dot_nn(w1_ref[...], h), 0.0) * mask
    h = lstm(u, wih1_ref, b1_ref, D0)
    u = _dot_nn(w2_ref[...], h) * mask
    out_ref[...] = lstm(u, wih2_ref, b2_ref, NCLS)


def _tc_specs():
    full = lambda shape: pl.BlockSpec(shape, lambda i: (0, 0))
    return [pl.BlockSpec((BLK, 128), lambda i: (i, 0)),
            pl.BlockSpec((NW, BLK // 128, 128), lambda i: (0, i, 0)),
            full((D0, 128)), full((4 * D0, D0)), full((D0, 4)),
            full((D0, D0)), full((4 * D0, D0)), full((D0, 4)),
            full((NCLS, D0)), full((4 * NCLS, NCLS)), full((NCLS, 4))]


def kernel(x, edge_index, W0, A0, Wih0, bih0, bhh0, W1, A1, Wih1, bih1, bhh1,
           W2, A2, Wih2, bih2, bhh2):
    marks = _sc_marks(edge_index.reshape(2 * E))
    mk = marks.reshape(NW, NPAD // 128, 128)

    out = pl.pallas_call(
        _tc_dense_body,
        grid=(pl.cdiv(N, BLK),),
        in_specs=_tc_specs(),
        out_specs=pl.BlockSpec((NCLS, BLK), lambda i: (0, i)),
        out_shape=jax.ShapeDtypeStruct((NCLS, N), jnp.float32),
    )(x, mk,
      W0.reshape(D0, 128), Wih0, (bih0 + bhh0).reshape(4, D0).T,
      W1.reshape(D0, D0), Wih1, (bih1 + bhh1).reshape(4, D0).T,
      W2.reshape(NCLS, D0), Wih2, (bih2 + bhh2).reshape(4, NCLS).T)
    return out.T
